# baseline (speedup 1.0000x reference)
import functools

import jax
import jax.numpy as jnp
from jax import lax
from jax.experimental import pallas as pl
from jax.experimental.pallas import tpu as pltpu
from jax.experimental.pallas import tpu_sc as plsc

_B, _T, _D, _K = 16, 1024, 64, 1024
_BETA = 0.25

_NC, _NS = 2, 16
_NW = _NC * _NS
_N = _B * _T
_CHUNK = 128
_NCH = _N // (_NW * _CHUNK)


def _vq_body(z_ref, cb_ref, nz_ref, ne_ref, sim_ref, ids_ref, loss_ref):
    i = pl.program_id(0)
    z = z_ref[...]
    cb = cb_ref[...]
    nz_row = nz_ref[pl.ds(i, 1), :]
    nz = jnp.transpose(nz_row, (1, 0))
    ne = ne_ref[...]
    dot = jax.lax.dot_general(z, cb, (((1,), (1,)), ((), ())),
                              preferred_element_type=jnp.float32)
    dist = -2.0 * dot + nz + ne
    m = jnp.min(dist, axis=1, keepdims=True)
    lane = jax.lax.broadcasted_iota(jnp.int32, (_T, _K), 1)
    idx = jnp.min(jnp.where(dist == m, lane, _K), axis=1, keepdims=True)
    sim_ref[...] = dot * jax.lax.rsqrt(nz) * jax.lax.rsqrt(ne)
    ids_ref[pl.ds(i, 1), :] = jnp.transpose(idx, (1, 0))
    norms = jnp.sqrt(jnp.maximum(m, 0.0))
    loss_ref[...] = jnp.sum(norms, axis=0, keepdims=True).reshape(1, 1)


@functools.partial(
    pl.kernel,
    mesh=plsc.VectorSubcoreMesh(core_axis_name="c", subcore_axis_name="s"),
    compiler_params=pltpu.CompilerParams(use_tc_tiling_on_sc=False),
    out_type=jax.ShapeDtypeStruct((_NW, _NCH, _CHUNK, _D), jnp.float32),
    scratch_types=[
        pltpu.VMEM((_NCH, _CHUNK), jnp.int32),
        pltpu.VMEM((_NCH, _CHUNK, _D), jnp.float32),
        pltpu.SemaphoreType.DMA,
    ],
)
def _sc_gather(cb_hbm, ids_hbm, out_hbm, idx_v, rows_v, sem):
    wid = lax.axis_index("s") * _NC + lax.axis_index("c")
    pltpu.sync_copy(ids_hbm.at[wid], idx_v)
    copies = [
        pltpu.async_copy(cb_hbm.at[idx_v.at[j]], rows_v.at[j], sem)
        for j in range(_NCH)
    ]
    for c in copies:
        c.wait()
    pltpu.sync_copy(rows_v, out_hbm.at[wid])


def kernel(z, codebook):
    nz2 = jnp.sum(jnp.square(z), axis=2)
    ne2 = jnp.sum(jnp.square(codebook), axis=1).reshape(1, _K)
    sim, ids, loss = pl.pallas_call(
        _vq_body,
        grid=(_B,),
        in_specs=[
            pl.BlockSpec((None, _T, _D), lambda i: (i, 0, 0)),
            pl.BlockSpec((_K, _D), lambda i: (0, 0)),
            pl.BlockSpec((_B, _T), lambda i: (0, 0)),
            pl.BlockSpec((1, _K), lambda i: (0, 0)),
        ],
        out_specs=[
            pl.BlockSpec((None, _T, _K), lambda i: (i, 0, 0)),
            pl.BlockSpec((_B, _T), lambda i: (0, 0)),
            pl.BlockSpec((None, 1, 1), lambda i: (i, 0, 0)),
        ],
        out_shape=[
            jax.ShapeDtypeStruct((_B, _T, _K), jnp.float32),
            jax.ShapeDtypeStruct((_B, _T), jnp.int32),
            jax.ShapeDtypeStruct((_B, 1, 1), jnp.float32),
        ],
    )(z, codebook, nz2, ne2)
    ids_chunks = ids.reshape(_NW, _NCH, _CHUNK)
    zq = _sc_gather(codebook, ids_chunks).reshape(_B, _T, _D)
    loss_vq = jnp.sum(loss) * (1.0 + _BETA) / (_B * _T)
    return zq, sim, ids, loss_vq
